# baseline (speedup 1.0000x reference)
import functools

import jax
import jax.numpy as jnp
from jax import lax
from jax.experimental import pallas as pl
from jax.experimental.pallas import tpu as pltpu
from jax.experimental.pallas import tpu_sc as plsc

N_NODES = 10000
N_EDGES = 320000
IN_DIM = 128
HID_DIM = 64
OUT_DIM = 32

NC = 2
NS = 16
NW = NC * NS
K = 128
TOT_CHUNKS = N_EDGES // K
BASE_CHUNKS = TOT_CHUNKS // NW
XTRA = TOT_CHUNKS - BASE_CHUNKS * NW
NPAIRS = BASE_CHUNKS // 2
EPW = BASE_CHUNKS * K
RPT = N_NODES // NS

_MESH = plsc.VectorSubcoreMesh(core_axis_name="c", subcore_axis_name="s")
_SC_PARAMS = pltpu.CompilerParams(use_tc_tiling_on_sc=False)


NBUF = 6
NGRP = BASE_CHUNKS // NBUF


def _make_deg_kernel():
    @functools.partial(
        pl.kernel,
        out_type=jax.ShapeDtypeStruct((N_NODES, 128), jnp.float32),
        mesh=_MESH,
        scratch_types=[
            pltpu.VMEM((EPW + K,), jnp.int32),
            pltpu.VMEM((K, 16), jnp.float32),
            pltpu.VMEM_SHARED((N_NODES, 16), jnp.float32),
            [pltpu.SemaphoreType.DMA] * NBUF,
        ],
        compiler_params=_SC_PARAMS,
    )
    def deg_kernel(e_hbm, ones_hbm, z_hbm, out_hbm, dst_v, ones_v, acc, ss):
        cid = lax.axis_index("c")
        sid = lax.axis_index("s")
        wid = cid * NS + sid
        pltpu.sync_copy(e_hbm.at[1, pl.ds(wid * EPW, EPW)],
                        dst_v.at[pl.ds(0, EPW)])
        pltpu.sync_copy(ones_hbm, ones_v)
        pltpu.sync_copy(z_hbm, acc.at[pl.ds(sid * RPT, RPT)])

        @pl.when(wid < XTRA)
        def _():
            pltpu.sync_copy(e_hbm.at[1, pl.ds(NW * EPW + wid * K, K)],
                            dst_v.at[pl.ds(EPW, K)])

        plsc.subcore_barrier()

        def scat(c, k):
            pltpu.async_copy(ones_v, acc.at[dst_v.at[pl.ds(c, K)]], ss[k],
                             add=True)

        def wait_scat(c, k):
            pltpu.make_async_copy(
                ones_v, acc.at[dst_v.at[pl.ds(c, K)]], ss[k]).wait()

        for k in range(NBUF):
            scat(k * K, k)

        @pl.loop(0, NGRP - 1)
        def _(i):
            q = i * (NBUF * K)
            for k in range(NBUF):
                wait_scat(q + k * K, k)
                scat(q + (NBUF + k) * K, k)

        base = (NGRP - 1) * NBUF * K
        for k in range(NBUF):
            wait_scat(base + k * K, k)

        @pl.when(wid < XTRA)
        def _():
            pltpu.sync_copy(ones_v, acc.at[dst_v.at[pl.ds(EPW, K)]], add=True)

        plsc.subcore_barrier()
        pltpu.sync_copy(acc.at[pl.ds(sid * RPT, RPT)],
                        out_hbm.at[pl.ds(sid * RPT, RPT), pl.ds(cid * 16, 16)])

    return deg_kernel


def _make_scatter_kernel(width):
    @functools.partial(
        pl.kernel,
        out_type=jax.ShapeDtypeStruct((N_NODES, 128), jnp.float32),
        mesh=_MESH,
        scratch_types=[
            pltpu.VMEM((EPW + K,), jnp.int32),
            pltpu.VMEM((EPW + K,), jnp.int32),
            [pltpu.VMEM((K, width), jnp.float32)] * NBUF,
            pltpu.VMEM_SHARED((N_NODES, width), jnp.float32),
            [pltpu.SemaphoreType.DMA] * NBUF,
            [pltpu.SemaphoreType.DMA] * NBUF,
        ],
        compiler_params=_SC_PARAMS,
    )
    def scat_kernel(y_hbm, e_hbm, z_hbm, out_hbm,
                    src_v, dst_v, bufs, acc, gs, ss):
        cid = lax.axis_index("c")
        sid = lax.axis_index("s")
        wid = cid * NS + sid
        pltpu.sync_copy(e_hbm.at[0, pl.ds(wid * EPW, EPW)],
                        src_v.at[pl.ds(0, EPW)])
        pltpu.sync_copy(e_hbm.at[1, pl.ds(wid * EPW, EPW)],
                        dst_v.at[pl.ds(0, EPW)])
        pltpu.sync_copy(z_hbm, acc.at[pl.ds(sid * RPT, RPT)])

        @pl.when(wid < XTRA)
        def _():
            pltpu.sync_copy(e_hbm.at[0, pl.ds(NW * EPW + wid * K, K)],
                            src_v.at[pl.ds(EPW, K)])
            pltpu.sync_copy(e_hbm.at[1, pl.ds(NW * EPW + wid * K, K)],
                            dst_v.at[pl.ds(EPW, K)])

        plsc.subcore_barrier()

        def gather(c, k):
            pltpu.async_copy(y_hbm.at[src_v.at[pl.ds(c, K)]], bufs[k], gs[k])

        def wait_gather(c, k):
            pltpu.make_async_copy(
                y_hbm.at[src_v.at[pl.ds(c, K)]], bufs[k], gs[k]).wait()

        def scatter(c, k):
            pltpu.async_copy(bufs[k], acc.at[dst_v.at[pl.ds(c, K)]], ss[k],
                             add=True)

        def wait_scatter(c, k):
            pltpu.make_async_copy(
                bufs[k], acc.at[dst_v.at[pl.ds(c, K)]], ss[k]).wait()

        for k in range(NBUF):
            gather(k * K, k)

        @pl.loop(0, NGRP)
        def _(i):
            q = i * (NBUF * K)
            for k in range(NBUF):
                wait_gather(q + k * K, k)
                scatter(q + k * K, k)
            for k in range(NBUF):
                nxt = q + (NBUF + k) * K

                @pl.when(nxt < BASE_CHUNKS * K)
                def _(k=k, q=q, nxt=nxt):
                    wait_scatter(q + k * K, k)
                    gather(nxt, k)

        base = NGRP * NBUF * K
        for k in range(NBUF):
            wait_scatter(base - (NBUF - k) * K, k)

        @pl.when(wid < XTRA)
        def _():
            pltpu.async_copy(y_hbm.at[src_v.at[pl.ds(EPW, K)]],
                             bufs[0], gs[0]).wait()
            pltpu.async_copy(bufs[0], acc.at[dst_v.at[pl.ds(EPW, K)]], ss[0],
                             add=True).wait()

        plsc.subcore_barrier()
        pltpu.sync_copy(
            acc.at[pl.ds(sid * RPT, RPT)],
            out_hbm.at[pl.ds(sid * RPT, RPT), pl.ds(cid * width, width)])

    return scat_kernel


_deg_kernel = _make_deg_kernel()
_scat64 = _make_scatter_kernel(HID_DIM)
_scat32 = _make_scatter_kernel(OUT_DIM)

_BM = 2000


def _tc1a_body(x_ref, w1_ref, xw_ref):
    xw_ref[...] = jnp.dot(x_ref[...], w1_ref[...],
                          preferred_element_type=jnp.float32)


def _tc1b_body(xw_ref, deg_ref, y_ref, p_ref):
    deg = deg_ref[:, 0:1] + deg_ref[:, 16:17] + 1.0
    dinv = lax.rsqrt(deg)
    xw = xw_ref[...]
    y_ref[...] = xw * dinv
    p_ref[...] = jnp.concatenate(
        [xw, jnp.broadcast_to(dinv, (_BM, IN_DIM - HID_DIM))], axis=1)


def _tc2_body(p_ref, s1_ref, b1_ref, w2_ref, y2_ref, q_ref):
    xw1 = p_ref[:, 0:HID_DIM]
    dinv = p_ref[:, HID_DIM:HID_DIM + 1]
    ssum = s1_ref[:, 0:HID_DIM] + s1_ref[:, HID_DIM:2 * HID_DIM]
    h = dinv * ssum + (dinv * dinv) * xw1
    h = jnp.maximum(h + b1_ref[0:1, :], 0.0)
    xw2 = jnp.dot(h, w2_ref[...], preferred_element_type=jnp.float32)
    y2_ref[...] = xw2 * dinv
    q_ref[...] = jnp.concatenate(
        [xw2, jnp.broadcast_to(dinv, (_BM, 128 - OUT_DIM))], axis=1)


def _tc3_body(q_ref, s2_ref, b2_ref, out_ref):
    xw2 = q_ref[:, 0:OUT_DIM]
    dinv = q_ref[:, OUT_DIM:OUT_DIM + 1]
    ssum = s2_ref[:, 0:OUT_DIM] + s2_ref[:, OUT_DIM:2 * OUT_DIM]
    out_ref[...] = dinv * ssum + (dinv * dinv) * xw2 + b2_ref[0:1, :]


def _specs(*shapes):
    specs = []
    for s in shapes:
        if len(s) == 2 and s[0] == N_NODES:
            specs.append(pl.BlockSpec((_BM, s[1]), lambda i: (i, 0)))
        else:
            specs.append(pl.BlockSpec(s, lambda i, s=s: tuple(0 for _ in s)))
    return specs


def kernel(x, edge_index, W1, b1, W2, b2):
    f32 = jnp.float32
    ei = edge_index.astype(jnp.int32)
    ones16 = jnp.ones((K, 16), f32)
    z16 = jnp.zeros((RPT, 16), f32)
    z64 = jnp.zeros((RPT, HID_DIM), f32)
    z32 = jnp.zeros((RPT, OUT_DIM), f32)
    b1_t = jnp.broadcast_to(b1.reshape(1, HID_DIM), (8, HID_DIM))
    b2_t = jnp.broadcast_to(b2.reshape(1, OUT_DIM), (8, OUT_DIM))

    grid = (N_NODES // _BM,)

    deg = _deg_kernel(ei, ones16, z16)

    xw1 = pl.pallas_call(
        _tc1a_body,
        grid=grid,
        in_specs=_specs((N_NODES, IN_DIM), (IN_DIM, HID_DIM)),
        out_specs=_specs((N_NODES, HID_DIM))[0],
        out_shape=jax.ShapeDtypeStruct((N_NODES, HID_DIM), f32),
    )(x, W1)

    y1, p = pl.pallas_call(
        _tc1b_body,
        grid=grid,
        in_specs=_specs((N_NODES, HID_DIM), (N_NODES, 128)),
        out_specs=_specs((N_NODES, HID_DIM), (N_NODES, 128)),
        out_shape=[jax.ShapeDtypeStruct((N_NODES, HID_DIM), f32),
                   jax.ShapeDtypeStruct((N_NODES, 128), f32)],
    )(xw1, deg)

    s1 = _scat64(y1, ei, z64)

    y2, q = pl.pallas_call(
        _tc2_body,
        grid=grid,
        in_specs=_specs((N_NODES, 128), (N_NODES, 2 * HID_DIM),
                        (8, HID_DIM), (HID_DIM, OUT_DIM)),
        out_specs=_specs((N_NODES, OUT_DIM), (N_NODES, 128)),
        out_shape=[jax.ShapeDtypeStruct((N_NODES, OUT_DIM), f32),
                   jax.ShapeDtypeStruct((N_NODES, 128), f32)],
    )(p, s1, b1_t, W2)

    s2 = _scat32(y2, ei, z32)

    out = pl.pallas_call(
        _tc3_body,
        grid=grid,
        in_specs=_specs((N_NODES, 128), (N_NODES, 128), (8, OUT_DIM)),
        out_specs=_specs((N_NODES, OUT_DIM))[0],
        out_shape=jax.ShapeDtypeStruct((N_NODES, OUT_DIM), f32),
    )(q, s2, b2_t)

    return out


# speedup vs baseline: 1.0052x; 1.0052x over previous
import functools

import jax
import jax.numpy as jnp
from jax import lax
from jax.experimental import pallas as pl
from jax.experimental.pallas import tpu as pltpu
from jax.experimental.pallas import tpu_sc as plsc

N_NODES = 10000
N_EDGES = 320000
IN_DIM = 128
HID_DIM = 64
OUT_DIM = 32

NC = 2
NS = 16
NW = NC * NS
K = 128
TOT_CHUNKS = N_EDGES // K
BASE_CHUNKS = TOT_CHUNKS // NW
XTRA = TOT_CHUNKS - BASE_CHUNKS * NW
NPAIRS = BASE_CHUNKS // 2
EPW = BASE_CHUNKS * K
RPT = N_NODES // NS

_MESH = plsc.VectorSubcoreMesh(core_axis_name="c", subcore_axis_name="s")
_SC_PARAMS = pltpu.CompilerParams(use_tc_tiling_on_sc=False)


NBUF = 6
NGRP = BASE_CHUNKS // NBUF


def _make_deg_kernel():
    @functools.partial(
        pl.kernel,
        out_type=jax.ShapeDtypeStruct((N_NODES, 128), jnp.float32),
        mesh=_MESH,
        scratch_types=[
            pltpu.VMEM((EPW + K,), jnp.int32),
            pltpu.VMEM((K, 16), jnp.float32),
            pltpu.VMEM_SHARED((N_NODES, 16), jnp.float32),
            [pltpu.SemaphoreType.DMA] * NBUF,
        ],
        compiler_params=_SC_PARAMS,
    )
    def deg_kernel(e_hbm, ones_hbm, z_hbm, out_hbm, dst_v, ones_v, acc, ss):
        cid = lax.axis_index("c")
        sid = lax.axis_index("s")
        wid = cid * NS + sid
        pltpu.sync_copy(e_hbm.at[1, pl.ds(wid * EPW, EPW)],
                        dst_v.at[pl.ds(0, EPW)])
        pltpu.sync_copy(ones_hbm, ones_v)
        pltpu.sync_copy(z_hbm, acc.at[pl.ds(sid * RPT, RPT)])

        @pl.when(wid < XTRA)
        def _():
            pltpu.sync_copy(e_hbm.at[1, pl.ds(NW * EPW + wid * K, K)],
                            dst_v.at[pl.ds(EPW, K)])

        plsc.subcore_barrier()

        def scat(c, k):
            pltpu.async_copy(ones_v, acc.at[dst_v.at[pl.ds(c, K)]], ss[k],
                             add=True)

        def wait_scat(c, k):
            pltpu.make_async_copy(
                ones_v, acc.at[dst_v.at[pl.ds(c, K)]], ss[k]).wait()

        for k in range(NBUF):
            scat(k * K, k)

        @pl.loop(0, NGRP - 1)
        def _(i):
            q = i * (NBUF * K)
            for k in range(NBUF):
                wait_scat(q + k * K, k)
                scat(q + (NBUF + k) * K, k)

        base = (NGRP - 1) * NBUF * K
        for k in range(NBUF):
            wait_scat(base + k * K, k)

        @pl.when(wid < XTRA)
        def _():
            pltpu.sync_copy(ones_v, acc.at[dst_v.at[pl.ds(EPW, K)]], add=True)

        plsc.subcore_barrier()
        pltpu.sync_copy(acc.at[pl.ds(sid * RPT, RPT)],
                        out_hbm.at[pl.ds(sid * RPT, RPT), pl.ds(cid * 16, 16)])

    return deg_kernel


def _make_scatter_kernel(width):
    @functools.partial(
        pl.kernel,
        out_type=jax.ShapeDtypeStruct((N_NODES, 128), jnp.float32),
        mesh=_MESH,
        scratch_types=[
            pltpu.VMEM((EPW + K,), jnp.int32),
            pltpu.VMEM((EPW + K,), jnp.int32),
            [pltpu.VMEM((K, width), jnp.float32)] * NBUF,
            pltpu.VMEM_SHARED((N_NODES, width), jnp.float32),
            [pltpu.SemaphoreType.DMA] * NBUF,
            [pltpu.SemaphoreType.DMA] * NBUF,
        ],
        compiler_params=_SC_PARAMS,
    )
    def scat_kernel(y_hbm, e_hbm, z_hbm, out_hbm,
                    src_v, dst_v, bufs, acc, gs, ss):
        cid = lax.axis_index("c")
        sid = lax.axis_index("s")
        wid = cid * NS + sid
        pltpu.sync_copy(e_hbm.at[0, pl.ds(wid * EPW, EPW)],
                        src_v.at[pl.ds(0, EPW)])
        pltpu.sync_copy(e_hbm.at[1, pl.ds(wid * EPW, EPW)],
                        dst_v.at[pl.ds(0, EPW)])
        pltpu.sync_copy(z_hbm, acc.at[pl.ds(sid * RPT, RPT)])

        @pl.when(wid < XTRA)
        def _():
            pltpu.sync_copy(e_hbm.at[0, pl.ds(NW * EPW + wid * K, K)],
                            src_v.at[pl.ds(EPW, K)])
            pltpu.sync_copy(e_hbm.at[1, pl.ds(NW * EPW + wid * K, K)],
                            dst_v.at[pl.ds(EPW, K)])

        plsc.subcore_barrier()

        def gather(c, k):
            pltpu.async_copy(y_hbm.at[src_v.at[pl.ds(c, K)]], bufs[k], gs[k])

        def wait_gather(c, k):
            pltpu.make_async_copy(
                y_hbm.at[src_v.at[pl.ds(c, K)]], bufs[k], gs[k]).wait()

        def scatter(c, k):
            pltpu.async_copy(bufs[k], acc.at[dst_v.at[pl.ds(c, K)]], ss[k],
                             add=True)

        def wait_scatter(c, k):
            pltpu.make_async_copy(
                bufs[k], acc.at[dst_v.at[pl.ds(c, K)]], ss[k]).wait()

        for k in range(NBUF):
            gather(k * K, k)

        @pl.loop(0, NGRP)
        def _(i):
            q = i * (NBUF * K)
            for k in range(NBUF):
                wait_gather(q + k * K, k)
                scatter(q + k * K, k)
            for k in range(NBUF):
                nxt = q + (NBUF + k) * K

                @pl.when(nxt < BASE_CHUNKS * K)
                def _(k=k, q=q, nxt=nxt):
                    wait_scatter(q + k * K, k)
                    gather(nxt, k)

        base = NGRP * NBUF * K
        for k in range(NBUF):
            wait_scatter(base - (NBUF - k) * K, k)

        @pl.when(wid < XTRA)
        def _():
            pltpu.async_copy(y_hbm.at[src_v.at[pl.ds(EPW, K)]],
                             bufs[0], gs[0]).wait()
            pltpu.async_copy(bufs[0], acc.at[dst_v.at[pl.ds(EPW, K)]], ss[0],
                             add=True).wait()

        plsc.subcore_barrier()
        pltpu.sync_copy(
            acc.at[pl.ds(sid * RPT, RPT)],
            out_hbm.at[pl.ds(sid * RPT, RPT), pl.ds(cid * width, width)])

    return scat_kernel


_deg_kernel = _make_deg_kernel()
_scat64 = _make_scatter_kernel(HID_DIM)
_scat32 = _make_scatter_kernel(OUT_DIM)

_BM = 2000


def _tc1_body(x_ref, w1_ref, deg_ref, y_ref, p_ref):
    deg = deg_ref[:, 0:1] + deg_ref[:, 16:17] + 1.0
    dinv = lax.rsqrt(deg)
    xw = jnp.dot(x_ref[...], w1_ref[...], preferred_element_type=jnp.float32)
    y_ref[...] = xw * dinv
    p_ref[...] = jnp.concatenate(
        [xw, jnp.broadcast_to(dinv, (_BM, IN_DIM - HID_DIM))], axis=1)


def _tc2_body(p_ref, s1_ref, b1_ref, w2_ref, y2_ref, q_ref):
    xw1 = p_ref[:, 0:HID_DIM]
    dinv = p_ref[:, HID_DIM:HID_DIM + 1]
    ssum = s1_ref[:, 0:HID_DIM] + s1_ref[:, HID_DIM:2 * HID_DIM]
    h = dinv * ssum + (dinv * dinv) * xw1
    h = jnp.maximum(h + b1_ref[0:1, :], 0.0)
    xw2 = jnp.dot(h, w2_ref[...], preferred_element_type=jnp.float32)
    y2_ref[...] = xw2 * dinv
    q_ref[...] = jnp.concatenate(
        [xw2, jnp.broadcast_to(dinv, (_BM, 128 - OUT_DIM))], axis=1)


def _tc3_body(q_ref, s2_ref, b2_ref, out_ref):
    xw2 = q_ref[:, 0:OUT_DIM]
    dinv = q_ref[:, OUT_DIM:OUT_DIM + 1]
    ssum = s2_ref[:, 0:OUT_DIM] + s2_ref[:, OUT_DIM:2 * OUT_DIM]
    out_ref[...] = dinv * ssum + (dinv * dinv) * xw2 + b2_ref[0:1, :]


def _specs(*shapes):
    specs = []
    for s in shapes:
        if len(s) == 2 and s[0] == N_NODES:
            specs.append(pl.BlockSpec((_BM, s[1]), lambda i: (i, 0)))
        else:
            specs.append(pl.BlockSpec(s, lambda i, s=s: tuple(0 for _ in s)))
    return specs


def kernel(x, edge_index, W1, b1, W2, b2):
    f32 = jnp.float32
    ei = edge_index.astype(jnp.int32)
    ones16 = jnp.ones((K, 16), f32)
    z16 = jnp.zeros((RPT, 16), f32)
    z64 = jnp.zeros((RPT, HID_DIM), f32)
    z32 = jnp.zeros((RPT, OUT_DIM), f32)
    b1_t = jnp.broadcast_to(b1.reshape(1, HID_DIM), (8, HID_DIM))
    b2_t = jnp.broadcast_to(b2.reshape(1, OUT_DIM), (8, OUT_DIM))

    grid = (N_NODES // _BM,)

    deg = _deg_kernel(ei, ones16, z16)

    y1, p = pl.pallas_call(
        _tc1_body,
        grid=grid,
        in_specs=_specs((N_NODES, IN_DIM), (IN_DIM, HID_DIM),
                        (N_NODES, 128)),
        out_specs=_specs((N_NODES, HID_DIM), (N_NODES, 128)),
        out_shape=[jax.ShapeDtypeStruct((N_NODES, HID_DIM), f32),
                   jax.ShapeDtypeStruct((N_NODES, 128), f32)],
    )(x, W1, deg)

    s1 = _scat64(y1, ei, z64)

    y2, q = pl.pallas_call(
        _tc2_body,
        grid=grid,
        in_specs=_specs((N_NODES, 128), (N_NODES, 2 * HID_DIM),
                        (8, HID_DIM), (HID_DIM, OUT_DIM)),
        out_specs=_specs((N_NODES, OUT_DIM), (N_NODES, 128)),
        out_shape=[jax.ShapeDtypeStruct((N_NODES, OUT_DIM), f32),
                   jax.ShapeDtypeStruct((N_NODES, 128), f32)],
    )(p, s1, b1_t, W2)

    s2 = _scat32(y2, ei, z32)

    out = pl.pallas_call(
        _tc3_body,
        grid=grid,
        in_specs=_specs((N_NODES, 128), (N_NODES, 128), (8, OUT_DIM)),
        out_specs=_specs((N_NODES, OUT_DIM))[0],
        out_shape=jax.ShapeDtypeStruct((N_NODES, OUT_DIM), f32),
    )(q, s2, b2_t)

    return out
